# baseline (speedup 1.0000x reference)
import functools

import jax
import jax.numpy as jnp
from jax import lax
from jax.experimental import pallas as pl
from jax.experimental.pallas import tpu as pltpu
from jax.experimental.pallas import tpu_sc as plsc

_N = 1024
_K = 20
_D = 3
_B = 16
_PAD = 16
_NW = 32
_ROWS_W = (_B * _N) // _NW


def _topk_body(xt_ref, x_ref, i_ref):
    b = pl.program_id(0)
    xb = x_ref[0]
    xtb = xt_ref[0]
    U = jnp.dot(xtb, xb, preferred_element_type=jnp.float32)

    r2 = jnp.sum(U * U, axis=0, keepdims=True)

    sq0 = r2 * jnp.float32(1.0 / (_N * _N))
    den0 = (1.0 + sq0) * jnp.sqrt(sq0 + 1e-12)
    b1 = sq0 * (r2 * jnp.float32(1.0 / _N)) / den0

    e1 = jnp.exp(b1 - jnp.max(b1, axis=1, keepdims=True))
    c1 = e1 / jnp.sum(e1, axis=1, keepdims=True)
    sq1 = c1 * c1 * r2
    den1 = (1.0 + sq1) * jnp.sqrt(sq1 + 1e-12)
    b2 = b1 + sq1 * (c1 * r2) / den1

    e2 = jnp.exp(b2 - jnp.max(b2, axis=1, keepdims=True))
    c2 = e2 / jnp.sum(e2, axis=1, keepdims=True)
    sq2 = c2 * c2 * r2
    s2 = c2 * U
    v2 = sq2 * s2 / ((1.0 + sq2) * jnp.sqrt(sq2 + 1e-12))

    bits = jax.lax.bitcast_convert_type(v2, jnp.int32)
    keys = jnp.where(bits >= 0, bits, bits ^ jnp.int32(0x7FFFFFFF))

    base = b * jnp.int32(_D * _N)
    iota_m = jax.lax.broadcasted_iota(jnp.int32, (_N, _N), 0)
    for j in range(_K):
        colmax = jnp.max(keys, axis=0, keepdims=True)
        cand = jnp.where(keys == colmax, iota_m, jnp.int32(_N))
        sel = jnp.min(cand, axis=0, keepdims=True)
        onehot = iota_m == sel
        keys = jnp.where(onehot, jnp.int32(-(2 ** 31)), keys)
        for d in range(_D):
            i_ref[0, d, j:j + 1, :] = sel + (base + jnp.int32(d * _N))


def _topk_indices(x, xt):
    return pl.pallas_call(
        _topk_body,
        grid=(_B,),
        in_specs=[
            pl.BlockSpec((1, _N, _D), lambda b: (b, 0, 0)),
            pl.BlockSpec((1, _D, _N), lambda b: (b, 0, 0)),
        ],
        out_specs=pl.BlockSpec((1, _D, _K, _N), lambda b: (b, 0, 0, 0)),
        out_shape=jax.ShapeDtypeStruct((_B, _D, _K, _N), jnp.int32),
    )(xt, x)


def _sc_gather(xflat, idxflat):
    mesh = plsc.VectorSubcoreMesh(core_axis_name="c", subcore_axis_name="s")

    @functools.partial(
        pl.kernel, mesh=mesh,
        out_type=jax.ShapeDtypeStruct((_B * 2 * _D * _K * _N,), jnp.float32),
        scratch_types=[
            pltpu.VMEM((_D * _K * _ROWS_W,), jnp.int32),
            pltpu.VMEM((_D * _K * _ROWS_W,), jnp.float32),
            pltpu.VMEM((_D * _K * _ROWS_W,), jnp.float32),
            pltpu.VMEM((_D * _ROWS_W,), jnp.float32),
            pltpu.SemaphoreType.DMA,
            pltpu.SemaphoreType.DMA,
            pltpu.SemaphoreType.DMA,
        ],
    )
    def k(xflat_hbm, idx_hbm, out_hbm, idx_a, gat_a, dif_a, ctr_v,
          sem_i, sem_g, sem_o):
        wid = lax.axis_index("s") * 2 + lax.axis_index("c")
        b = wid // 2
        n0 = (wid % 2) * _ROWS_W
        nj = _D * _K

        for d in range(_D):
            pltpu.async_copy(
                xflat_hbm.at[pl.ds((b * _D + d) * _N + n0, _ROWS_W)],
                ctr_v.at[pl.ds(d * _ROWS_W, _ROWS_W)], sem_i)

        def start_idx(t, _):
            d = t // _K
            j = t - d * _K
            off = ((b * _D + d) * _K + j) * _N + n0
            pltpu.async_copy(idx_hbm.at[pl.ds(off, _ROWS_W)],
                             idx_a.at[pl.ds(t * _ROWS_W, _ROWS_W)], sem_i)
            return 0

        lax.fori_loop(0, nj, start_idx, 0)

        def wait_i(t, _):
            pltpu.make_async_copy(
                idx_hbm.at[pl.ds(0, _ROWS_W)],
                idx_a.at[pl.ds(0, _ROWS_W)], sem_i).wait()
            return 0

        lax.fori_loop(0, nj + _D, wait_i, 0)

        def start_gat(t, _):
            pltpu.async_copy(
                xflat_hbm.at[idx_a.at[pl.ds(t * _ROWS_W, _ROWS_W)]],
                gat_a.at[pl.ds(t * _ROWS_W, _ROWS_W)], sem_g)
            return 0

        lax.fori_loop(0, nj, start_gat, 0)

        def wait_g(t, _):
            pltpu.make_async_copy(
                xflat_hbm.at[pl.ds(0, _ROWS_W)],
                gat_a.at[pl.ds(0, _ROWS_W)], sem_g).wait()
            return 0

        lax.fori_loop(0, nj, wait_g, 0)

        def sub(i, _):
            t = i // (_ROWS_W // 16)
            d = t // _K
            r = i - t * (_ROWS_W // 16)
            s = pl.ds(i * 16, 16)
            c = pl.ds(d * _ROWS_W + r * 16, 16)
            dif_a[s] = gat_a[s] - ctr_v[c]
            return 0

        lax.fori_loop(0, nj * (_ROWS_W // 16), sub, 0)

        def start_out(t, _):
            d = t // _K
            j = t - d * _K
            off_d = ((b * 2 * _D + d) * _K + j) * _N + n0
            off_f = ((b * 2 * _D + _D + d) * _K + j) * _N + n0
            src = pl.ds(t * _ROWS_W, _ROWS_W)
            pltpu.async_copy(dif_a.at[src],
                             out_hbm.at[pl.ds(off_d, _ROWS_W)], sem_o)
            pltpu.async_copy(gat_a.at[src],
                             out_hbm.at[pl.ds(off_f, _ROWS_W)], sem_o)
            return 0

        lax.fori_loop(0, nj, start_out, 0)

        def wait_o(t, _):
            pltpu.make_async_copy(
                dif_a.at[pl.ds(0, _ROWS_W)],
                out_hbm.at[pl.ds(0, _ROWS_W)], sem_o).wait()
            return 0

        lax.fori_loop(0, 2 * nj, wait_o, 0)

    return k(xflat, idxflat)


def kernel(x, l):
    del l
    xt = jnp.transpose(x, (0, 2, 1))
    idx = _topk_indices(x, xt)
    out_flat = _sc_gather(x.reshape(-1), idx.reshape(-1))
    out_t = out_flat.reshape(_B, 2 * _D, _K, _N)
    return jnp.transpose(out_t, (0, 1, 3, 2))


# speedup vs baseline: 1.0016x; 1.0016x over previous
import functools

import jax
import jax.numpy as jnp
from jax import lax
from jax.experimental import pallas as pl
from jax.experimental.pallas import tpu as pltpu
from jax.experimental.pallas import tpu_sc as plsc

_N = 1024
_K = 20
_D = 3
_B = 16
_PAD = 16
_NW = 32
_ROWS_W = (_B * _N) // _NW


def _topk_body(xt_ref, x_ref, i_ref):
    b = pl.program_id(0)
    xb = x_ref[0]
    xtb = xt_ref[0]
    U = jnp.dot(xtb, xb, preferred_element_type=jnp.float32)

    r2 = jnp.sum(U * U, axis=0, keepdims=True)

    sq0 = r2 * jnp.float32(1.0 / (_N * _N))
    den0 = (1.0 + sq0) * jnp.sqrt(sq0 + 1e-12)
    b1 = sq0 * (r2 * jnp.float32(1.0 / _N)) / den0

    e1 = jnp.exp(b1 - jnp.max(b1, axis=1, keepdims=True))
    c1 = e1 / jnp.sum(e1, axis=1, keepdims=True)
    sq1 = c1 * c1 * r2
    den1 = (1.0 + sq1) * jnp.sqrt(sq1 + 1e-12)
    b2 = b1 + sq1 * (c1 * r2) / den1

    e2 = jnp.exp(b2 - jnp.max(b2, axis=1, keepdims=True))
    c2 = e2 / jnp.sum(e2, axis=1, keepdims=True)
    sq2 = c2 * c2 * r2
    s2 = c2 * U
    v2 = sq2 * s2 / ((1.0 + sq2) * jnp.sqrt(sq2 + 1e-12))

    bits = jax.lax.bitcast_convert_type(v2, jnp.int32)
    keys = jnp.where(bits >= 0, bits, bits ^ jnp.int32(0x7FFFFFFF))

    base = b * jnp.int32(_D * _N)
    iota_m = jax.lax.broadcasted_iota(jnp.int32, (_N, _N), 0)
    for j in range(_K):
        colmax = jnp.max(keys, axis=0, keepdims=True)
        cand = jnp.where(keys == colmax, iota_m, jnp.int32(_N))
        sel = jnp.min(cand, axis=0, keepdims=True)
        onehot = iota_m == sel
        keys = jnp.where(onehot, jnp.int32(-(2 ** 31)), keys)
        for d in range(_D):
            i_ref[0, d, j:j + 1, :] = sel + (base + jnp.int32(d * _N))


def _topk_indices(x, xt):
    return pl.pallas_call(
        _topk_body,
        grid=(_B,),
        in_specs=[
            pl.BlockSpec((1, _N, _D), lambda b: (b, 0, 0)),
            pl.BlockSpec((1, _D, _N), lambda b: (b, 0, 0)),
        ],
        out_specs=pl.BlockSpec((1, _D, _K, _N), lambda b: (b, 0, 0, 0)),
        out_shape=jax.ShapeDtypeStruct((_B, _D, _K, _N), jnp.int32),
    )(xt, x)


def _sc_gather(xflat, idxflat):
    mesh = plsc.VectorSubcoreMesh(core_axis_name="c", subcore_axis_name="s")

    @functools.partial(
        pl.kernel, mesh=mesh,
        out_type=jax.ShapeDtypeStruct((_B * 2 * _D * _K * _N,), jnp.float32),
        scratch_types=[
            pltpu.VMEM((_D * _K * _ROWS_W,), jnp.int32),
            pltpu.VMEM((_D * _K * _ROWS_W,), jnp.float32),
            pltpu.VMEM((_D * _K * _ROWS_W,), jnp.float32),
            pltpu.VMEM((_D * _ROWS_W,), jnp.float32),
            pltpu.SemaphoreType.DMA,
            pltpu.SemaphoreType.DMA,
            pltpu.SemaphoreType.DMA,
        ],
    )
    def k(xflat_hbm, idx_hbm, out_hbm, idx_a, gat_a, dif_a, ctr_v,
          sem_i, sem_g, sem_o):
        wid = lax.axis_index("s") * 2 + lax.axis_index("c")
        b = wid // 2
        n0 = (wid % 2) * _ROWS_W
        nj = _D * _K

        for d in range(_D):
            pltpu.async_copy(
                xflat_hbm.at[pl.ds((b * _D + d) * _N + n0, _ROWS_W)],
                ctr_v.at[pl.ds(d * _ROWS_W, _ROWS_W)], sem_i)

        def start_idx(t, _):
            d = t // _K
            j = t - d * _K
            off = ((b * _D + d) * _K + j) * _N + n0
            pltpu.async_copy(idx_hbm.at[pl.ds(off, _ROWS_W)],
                             idx_a.at[pl.ds(t * _ROWS_W, _ROWS_W)], sem_i)
            return 0

        lax.fori_loop(0, nj, start_idx, 0)

        def wait_i(t, _):
            pltpu.make_async_copy(
                idx_hbm.at[pl.ds(0, _ROWS_W)],
                idx_a.at[pl.ds(0, _ROWS_W)], sem_i).wait()
            return 0

        lax.fori_loop(0, nj + _D, wait_i, 0)

        def start_gat(t, _):
            pltpu.async_copy(
                xflat_hbm.at[idx_a.at[pl.ds(t * _ROWS_W, _ROWS_W)]],
                gat_a.at[pl.ds(t * _ROWS_W, _ROWS_W)], sem_g)
            return 0

        lax.fori_loop(0, nj, start_gat, 0)

        def wait_g(t, _):
            pltpu.make_async_copy(
                xflat_hbm.at[pl.ds(0, _ROWS_W)],
                gat_a.at[pl.ds(0, _ROWS_W)], sem_g).wait()
            return 0

        lax.fori_loop(0, nj, wait_g, 0)

        for d in range(_D):
            def sub(i, _, d=d):
                s = pl.ds(d * _K * _ROWS_W + i * 16, 16)
                r = i & (_ROWS_W // 16 - 1)
                c = pl.ds(d * _ROWS_W + r * 16, 16)
                dif_a[s] = gat_a[s] - ctr_v[c]
                return 0

            lax.fori_loop(0, _K * (_ROWS_W // 16), sub, 0)

        def start_out(t, _):
            d = t // _K
            j = t - d * _K
            off_d = ((b * 2 * _D + d) * _K + j) * _N + n0
            off_f = ((b * 2 * _D + _D + d) * _K + j) * _N + n0
            src = pl.ds(t * _ROWS_W, _ROWS_W)
            pltpu.async_copy(dif_a.at[src],
                             out_hbm.at[pl.ds(off_d, _ROWS_W)], sem_o)
            pltpu.async_copy(gat_a.at[src],
                             out_hbm.at[pl.ds(off_f, _ROWS_W)], sem_o)
            return 0

        lax.fori_loop(0, nj, start_out, 0)

        def wait_o(t, _):
            pltpu.make_async_copy(
                dif_a.at[pl.ds(0, _ROWS_W)],
                out_hbm.at[pl.ds(0, _ROWS_W)], sem_o).wait()
            return 0

        lax.fori_loop(0, 2 * nj, wait_o, 0)

    return k(xflat, idxflat)


def kernel(x, l):
    del l
    xt = jnp.transpose(x, (0, 2, 1))
    idx = _topk_indices(x, xt)
    out_flat = _sc_gather(x.reshape(-1), idx.reshape(-1))
    out_t = out_flat.reshape(_B, 2 * _D, _K, _N)
    return jnp.transpose(out_t, (0, 1, 3, 2))


# speedup vs baseline: 1.2285x; 1.2265x over previous
import functools

import jax
import jax.numpy as jnp
from jax import lax
from jax.experimental import pallas as pl
from jax.experimental.pallas import tpu as pltpu
from jax.experimental.pallas import tpu_sc as plsc

_N = 1024
_K = 20
_D = 3
_B = 16
_PAD = 16
_NW = 32
_ROWS_W = (_B * _N) // _NW


def _topk_body(xt_ref, x_ref, i_ref):
    b = pl.program_id(0)
    xb = x_ref[0]
    xtb = xt_ref[0]
    U = jnp.dot(xtb, xb, preferred_element_type=jnp.float32)

    r2 = jnp.sum(U * U, axis=0, keepdims=True)

    sq0 = r2 * jnp.float32(1.0 / (_N * _N))
    den0 = (1.0 + sq0) * jnp.sqrt(sq0 + 1e-12)
    b1 = sq0 * (r2 * jnp.float32(1.0 / _N)) / den0

    e1 = jnp.exp(b1 - jnp.max(b1, axis=1, keepdims=True))
    c1 = e1 / jnp.sum(e1, axis=1, keepdims=True)
    sq1 = c1 * c1 * r2
    den1 = (1.0 + sq1) * jnp.sqrt(sq1 + 1e-12)
    b2 = b1 + sq1 * (c1 * r2) / den1

    e2 = jnp.exp(b2 - jnp.max(b2, axis=1, keepdims=True))
    c2 = e2 / jnp.sum(e2, axis=1, keepdims=True)
    sq2 = c2 * c2 * r2
    s2 = c2 * U
    v2 = sq2 * s2 / ((1.0 + sq2) * jnp.sqrt(sq2 + 1e-12))

    bits = jax.lax.bitcast_convert_type(v2, jnp.int32)
    keys = jnp.where(bits >= 0, bits, bits ^ jnp.int32(0x7FFFFFFF))

    base = b * jnp.int32(_D * _N)
    iota_m = jax.lax.broadcasted_iota(jnp.int32, (_N, _N), 0)
    for j in range(_K):
        colmax = jnp.max(keys, axis=0, keepdims=True)
        cand = jnp.where(keys == colmax, iota_m, jnp.int32(_N))
        sel = jnp.min(cand, axis=0, keepdims=True)
        onehot = iota_m == sel
        keys = jnp.where(onehot, jnp.int32(-(2 ** 31)), keys)
        for d in range(_D):
            i_ref[0, d, j:j + 1, :] = sel + (base + jnp.int32(d * _N))


def _topk_indices(x, xt):
    return pl.pallas_call(
        _topk_body,
        grid=(_B,),
        in_specs=[
            pl.BlockSpec((1, _N, _D), lambda b: (b, 0, 0)),
            pl.BlockSpec((1, _D, _N), lambda b: (b, 0, 0)),
        ],
        out_specs=pl.BlockSpec((1, _D, _K, _N), lambda b: (b, 0, 0, 0)),
        out_shape=jax.ShapeDtypeStruct((_B, _D, _K, _N), jnp.int32),
    )(xt, x)


def _sc_gather(xflat, idxflat):
    mesh = plsc.VectorSubcoreMesh(core_axis_name="c", subcore_axis_name="s")

    @functools.partial(
        pl.kernel, mesh=mesh,
        out_type=jax.ShapeDtypeStruct((_B * 2 * _D * _K * _N,), jnp.float32),
        scratch_types=[
            pltpu.VMEM((_D * _ROWS_W,), jnp.int32),
            pltpu.VMEM((_D * _ROWS_W,), jnp.float32),
            pltpu.VMEM((_D * _ROWS_W,), jnp.float32),
            pltpu.VMEM((_D * _ROWS_W,), jnp.float32),
            pltpu.SemaphoreType.DMA,
            pltpu.SemaphoreType.DMA,
            pltpu.SemaphoreType.DMA,
        ],
    )
    def k(xflat_hbm, idx_hbm, out_hbm, idx_a, gat_a, dif_a, ctr_v,
          sem_i, sem_g, sem_o):
        wid = lax.axis_index("s") * 2 + lax.axis_index("c")
        b = wid // 2
        n0 = (wid % 2) * _ROWS_W

        def lst(buf, d):
            return buf.at[pl.ds(d * _ROWS_W, _ROWS_W)]

        for d in range(_D):
            pltpu.sync_copy(
                xflat_hbm.at[pl.ds((b * _D + d) * _N + n0, _ROWS_W)],
                lst(ctr_v, d))

        for j in range(_K):
            hi = []
            for d in range(_D):
                off = ((b * _D + d) * _K + j) * _N + n0
                hi.append(pltpu.async_copy(
                    idx_hbm.at[pl.ds(off, _ROWS_W)], lst(idx_a, d), sem_i))
            for h in hi:
                h.wait()
            if j > 0:
                for h in ho:
                    h.wait()
            hg = []
            for d in range(_D):
                hg.append(pltpu.async_copy(
                    xflat_hbm.at[lst(idx_a, d)], lst(gat_a, d), sem_g))
            for h in hg:
                h.wait()

            def sub(i, _):
                s = pl.ds(i * 16, 16)
                dif_a[s] = gat_a[s] - ctr_v[s]
                return 0

            lax.fori_loop(0, _D * _ROWS_W // 16, sub, 0)

            ho = []
            for d in range(_D):
                off_d = ((b * 2 * _D + d) * _K + j) * _N + n0
                off_f = ((b * 2 * _D + _D + d) * _K + j) * _N + n0
                ho.append(pltpu.async_copy(
                    lst(dif_a, d), out_hbm.at[pl.ds(off_d, _ROWS_W)], sem_o))
                ho.append(pltpu.async_copy(
                    lst(gat_a, d), out_hbm.at[pl.ds(off_f, _ROWS_W)], sem_o))
        for h in ho:
            h.wait()

    return k(xflat, idxflat)


def kernel(x, l):
    del l
    xt = jnp.transpose(x, (0, 2, 1))
    idx = _topk_indices(x, xt)
    out_flat = _sc_gather(x.reshape(-1), idx.reshape(-1))
    out_t = out_flat.reshape(_B, 2 * _D, _K, _N)
    return jnp.transpose(out_t, (0, 1, 3, 2))
